# Initial kernel scaffold; baseline (speedup 1.0000x reference)
import jax
import jax.numpy as jnp
from jax.experimental import pallas as pl


def kernel(bins, weights, n_samples):
    raise NotImplementedError("write your pallas kernel here")


# speedup vs baseline: 8.0652x; 8.0652x over previous
import functools

import jax
import jax.numpy as jnp
from jax import lax
from jax.experimental import pallas as pl
from jax.experimental.pallas import tpu as pltpu
from jax.experimental.pallas import tpu_sc as plsc

N_RAYS = 65536
N_W = 254
N_BINS = 255
N_S = 256
L = 16

NC = 2
NSUB = 16
NWORK = NC * NSUB
ROWS_PER_W = N_RAYS // NWORK
R = 32
NCHUNK = ROWS_PER_W // R

HIST_PAD = 272


def _sc_body(wf, bf, out, w0, w1, b0, b1, o0, o1, excl, hist,
             sw0, sw1, sb0, sb1, so0, so1):
    cid = lax.axis_index("c")
    sid = lax.axis_index("s")
    wid = sid * NC + cid
    row0 = wid * ROWS_PER_W

    wbufs = (w0, w1)
    bbufs = (b0, b1)
    obufs = (o0, o1)
    swse = (sw0, sw1)
    sbse = (sb0, sb1)
    sose = (so0, so1)

    def in_copies(chunk, slot):
        base = row0 + chunk * R
        cw = pltpu.make_async_copy(
            wf.at[pl.ds(base * N_W, R * N_W)],
            wbufs[slot].at[pl.ds(0, R * N_W)], swse[slot])
        cb = pltpu.make_async_copy(
            bf.at[pl.ds(base * N_BINS, R * N_BINS)],
            bbufs[slot], sbse[slot])
        return cw, cb

    def out_copy(chunk, slot):
        base = row0 + chunk * R
        return pltpu.make_async_copy(
            obufs[slot], out.at[pl.ds(base * N_S, R * N_S)], sose[slot])

    lane = lax.iota(jnp.int32, L)
    lane_f = lane.astype(jnp.float32)
    tail_mask = lane < (N_W - 15 * L)
    ones_i = jnp.ones((L,), jnp.int32)
    zeros_i = jnp.zeros((L,), jnp.int32)

    def compute_chunk(slot):
        wbuf = wbufs[slot]
        bbuf = bbufs[slot]
        obuf = obufs[slot]

        def ray_body(rr, acc):
            wb = rr * N_W
            carry = jnp.float32(0.0)
            for c in range(16):
                v = wbuf[pl.ds(wb + c * L, L)]
                if c == 15:
                    v = jnp.where(tail_mask, v + 1e-5, 0.0)
                else:
                    v = v + 1e-5
                s = plsc.cumsum(v)
                excl[pl.ds(c * L, L)] = s - v + carry
                carry = carry + jnp.sum(v)
            inv = 1.0 / jnp.full((L,), carry, jnp.float32)
            inv256 = 256.0 * inv

            for c in range(17):
                hist[pl.ds(c * L, L)] = zeros_i
            for c in range(16):
                e = excl[pl.ds(c * L, L)]
                y = e * inv256 - 0.5
                y0 = jnp.maximum(y, 0.0)
                ti = y0.astype(jnp.int32)
                st = ti + jnp.where(ti.astype(jnp.float32) < y0, 1, 0)
                st = jnp.minimum(st, 256)
                plsc.addupdate_scatter(hist, [st], ones_i)

            bb = rr * N_BINS
            ob = rr * N_S
            ci = jnp.int32(0)
            for jc in range(16):
                h = hist[pl.ds(jc * L, L)]
                inds = plsc.cumsum(h) + ci
                ci = ci + jnp.sum(h)
                below = jnp.maximum(inds - 1, 0)
                above = jnp.minimum(inds, N_BINS - 1)
                e0 = plsc.load_gather(excl, [below])
                e1 = plsc.load_gather(excl, [above])
                g0 = plsc.load_gather(bbuf, [bb + below])
                g1 = plsc.load_gather(bbuf, [bb + above])
                u = (lane_f + (jc * L + 0.5)) * (1.0 / 256.0)
                c0 = e0 * inv
                den = (e1 - e0) * inv
                den = jnp.where(den < 1e-5, 1.0, den)
                t = (u - c0) / den
                obuf[pl.ds(ob + jc * L, L)] = g0 + t * (g1 - g0)
            return acc

        lax.fori_loop(0, R, ray_body, jnp.int32(0))

    for s in range(2):
        cw, cb = in_copies(s, s)
        cw.start()
        cb.start()

    def step(g, acc):
        for slot in range(2):
            i = 2 * g + slot
            cw, cb = in_copies(i, slot)
            cw.wait()
            cb.wait()

            @pl.when(g > 0)
            def _():
                out_copy(i - 2, slot).wait()

            compute_chunk(slot)
            out_copy(i, slot).start()

            @pl.when(i + 2 < NCHUNK)
            def _():
                cw2, cb2 = in_copies(i + 2, slot)
                cw2.start()
                cb2.start()
        return acc

    lax.fori_loop(0, NCHUNK // 2, step, jnp.int32(0))
    out_copy(NCHUNK - 2, 0).wait()
    out_copy(NCHUNK - 1, 1).wait()


def _sample_pdf(bins, weights):
    assert bins.shape == (N_RAYS, N_BINS)
    wf = weights.reshape(-1)
    bf = bins.reshape(-1)

    mesh = plsc.VectorSubcoreMesh(core_axis_name="c", subcore_axis_name="s")
    run = functools.partial(
        pl.kernel,
        out_type=jax.ShapeDtypeStruct((N_RAYS * N_S,), jnp.float32),
        mesh=mesh,
        compiler_params=pltpu.CompilerParams(needs_layout_passes=False),
        scratch_types=[
            pltpu.VMEM((R * N_W + L,), jnp.float32),
            pltpu.VMEM((R * N_W + L,), jnp.float32),
            pltpu.VMEM((R * N_BINS,), jnp.float32),
            pltpu.VMEM((R * N_BINS,), jnp.float32),
            pltpu.VMEM((R * N_S,), jnp.float32),
            pltpu.VMEM((R * N_S,), jnp.float32),
            pltpu.VMEM((N_S,), jnp.float32),
            pltpu.VMEM((HIST_PAD,), jnp.int32),
            pltpu.SemaphoreType.DMA,
            pltpu.SemaphoreType.DMA,
            pltpu.SemaphoreType.DMA,
            pltpu.SemaphoreType.DMA,
            pltpu.SemaphoreType.DMA,
            pltpu.SemaphoreType.DMA,
        ],
    )(_sc_body)
    outf = run(wf, bf)
    return outf.reshape(N_RAYS, N_S)


def kernel(bins, weights, n_samples):
    del n_samples
    return _sample_pdf(bins, weights)
